# baseline (speedup 1.0000x reference)
import functools

import jax
import jax.numpy as jnp
from jax import lax
from jax.experimental import pallas as pl
from jax.experimental.pallas import tpu as pltpu
from jax.experimental.pallas import tpu_sc as plsc

N = 100000
E = 3200000
OUT_DIM = 128
NUM_TYPES = 16

NW = 32
CHUNK = 512
JJ = CHUNK // 128
EROWS = E // 128
NCHT = E // CHUNK
EROWB = 1568
EROWS_PAD = 16 * EROWB
PAIRS = NCHT // 2
BASE_PAIRS = PAIRS // NW
EXTRA_PAIRS = PAIRS - BASE_PAIRS * NW

NPAD = 112000
RPT = NPAD // 16
ZR = 56
NZC = RPT // ZR

RB1 = 6400
GRID1 = -(-N // RB1)
NP8 = NPAD // 8
RB3P = 560
RB3 = RB3P * 8
GRID3 = -(-(N // 8) // RB3P)


def _p_body(r_ref, w1_ref, b1_ref, w2s_ref, b2s_ref, g_ref, ei_ref,
            p_ref, eo_ref):
    eo_ref[...] = ei_ref[...].reshape(2, EROWB, 128)
    z = jnp.maximum(
        jnp.dot(r_ref[...], w1_ref[...], preferred_element_type=jnp.float32)
        + b1_ref[...], 0.0)
    zp = z.reshape(RB1 // 8, 8 * OUT_DIM)
    l = (jnp.dot(zp, w2s_ref[...], preferred_element_type=jnp.float32)
         + b2s_ref[...])
    m = jnp.max(l, axis=-1, keepdims=True)
    e = jnp.exp(jnp.maximum(l - m, -60.0))
    s = jnp.dot(e, g_ref[...], preferred_element_type=jnp.float32)
    p_ref[...] = e / s


def _out_body(a0_ref, a1_ref, g_ref, w3s_ref, b3t_ref, o_ref):
    sums = a0_ref[...] + a1_ref[...]
    rs = jnp.dot(sums, g_ref[...], preferred_element_type=jnp.float32)
    inv = jnp.where(rs > 0.5, 1.0 / jnp.maximum(rs, 0.5), 0.0)
    nd = sums * inv
    res = jnp.maximum(
        jnp.dot(nd, w3s_ref[...], preferred_element_type=jnp.float32)
        + b3t_ref[...], 0.0)
    o_ref[...] = res.reshape(RB3, 128)


_compute_p = pl.pallas_call(
    _p_body,
    grid=(GRID1,),
    in_specs=[
        pl.BlockSpec((RB1, NUM_TYPES), lambda i: (i, 0)),
        pl.BlockSpec((NUM_TYPES, OUT_DIM), lambda i: (0, 0)),
        pl.BlockSpec((1, OUT_DIM), lambda i: (0, 0)),
        pl.BlockSpec((8 * OUT_DIM, 8 * NUM_TYPES), lambda i: (0, 0)),
        pl.BlockSpec((1, 8 * NUM_TYPES), lambda i: (0, 0)),
        pl.BlockSpec((8 * NUM_TYPES, 8 * NUM_TYPES), lambda i: (0, 0)),
        pl.BlockSpec((2, EROWB * 128), lambda i: (0, i)),
    ],
    out_specs=[
        pl.BlockSpec((RB1 // 8, 8 * NUM_TYPES), lambda i: (i, 0)),
        pl.BlockSpec((2, EROWB, 128), lambda i: (0, i, 0)),
    ],
    out_shape=[
        jax.ShapeDtypeStruct((N // 8, 8 * NUM_TYPES), jnp.float32),
        jax.ShapeDtypeStruct((2, EROWS_PAD, 128), jnp.int32),
    ],
)

_compute_out = pl.pallas_call(
    _out_body,
    grid=(GRID3,),
    in_specs=[
        pl.BlockSpec((RB3P, 128), lambda i: (i, 0)),
        pl.BlockSpec((RB3P, 128), lambda i: (i + NPAD // RB3, 0)),
        pl.BlockSpec((128, 128), lambda i: (0, 0)),
        pl.BlockSpec((128, 8 * OUT_DIM), lambda i: (0, 0)),
        pl.BlockSpec((1, 8 * OUT_DIM), lambda i: (0, 0)),
    ],
    out_specs=pl.BlockSpec((RB3, OUT_DIM), lambda i: (i, 0)),
    out_shape=jax.ShapeDtypeStruct((N, OUT_DIM), jnp.float32),
)


@functools.partial(
    pl.kernel,
    out_type=jax.ShapeDtypeStruct((2 * NPAD, NUM_TYPES), jnp.float32),
    mesh=plsc.VectorSubcoreMesh(core_axis_name="c", subcore_axis_name="s"),
    scratch_types=[
        pltpu.VMEM_SHARED((NPAD, NUM_TYPES), jnp.float32),
        pltpu.VMEM((JJ, 128), jnp.int32),
        pltpu.VMEM((JJ, 128), jnp.int32),
        pltpu.VMEM((JJ, 128), jnp.int32),
        pltpu.VMEM((JJ, 128), jnp.int32),
        pltpu.VMEM((CHUNK, NUM_TYPES), jnp.float32),
        pltpu.VMEM((CHUNK, NUM_TYPES), jnp.float32),
        pltpu.SemaphoreType.DMA,
        pltpu.SemaphoreType.DMA,
        pltpu.SemaphoreType.DMA,
        pltpu.SemaphoreType.DMA,
    ],
    compiler_params=pltpu.CompilerParams(use_tc_tiling_on_sc=False),
)
def _sc_segsum(p_hbm, ei_hbm, out_hbm,
               acc_sh, s0, s1, d0, d1, rb0, rb1,
               gsem0, gsem1, isem, ssem):
    c = lax.axis_index("c")
    s = lax.axis_index("s")
    wid = c * 16 + s
    S = (s0, s1)
    D = (d0, d1)
    RBUF = (rb0, rb1)
    GSEM = (gsem0, gsem1)

    zv = jnp.zeros((NUM_TYPES,), jnp.float32)
    for i in range(ZR):
        rb0[i] = zv
    row0 = s * RPT

    def zbody(k, carry):
        pltpu.sync_copy(rb0.at[pl.ds(0, ZR)],
                        acc_sh.at[pl.ds(row0 + k * ZR, ZR)])
        return carry
    lax.fori_loop(0, NZC, zbody, 0)
    plsc.subcore_barrier()

    pair0 = wid * BASE_PAIRS + jnp.minimum(wid, EXTRA_PAIRS)
    npairs = jnp.where(wid < EXTRA_PAIRS, BASE_PAIRS + 1, BASE_PAIRS)
    wrow = pair0 * 2 * JJ
    rmax = EROWS - JJ

    def gather_copy(b, j):
        return pltpu.make_async_copy(
            p_hbm.at[S[b].at[j]],
            RBUF[b].at[pl.ds(j * 128, 128)], GSEM[b])

    def scatter_copy(b, j):
        return pltpu.make_async_copy(
            RBUF[b].at[pl.ds(j * 128, 128)],
            acc_sh.at[D[b].at[j]], ssem)

    pltpu.sync_copy(ei_hbm.at[0, pl.ds(wrow, JJ)], s0)
    for j in range(JJ):
        gather_copy(0, j).start()
    pltpu.async_copy(ei_hbm.at[0, pl.ds(wrow + JJ, JJ)], s1, isem)
    pltpu.async_copy(ei_hbm.at[1, pl.ds(wrow, JJ)], d0, isem)

    def pair_body(u, carry):
        for b in range(2):
            t = u * 2 + b
            rbase = wrow + t * JJ
            for j in range(JJ):
                gather_copy(b, j).wait()

            @pl.when(t > 0)
            def _():
                for j in range(JJ):
                    scatter_copy(b ^ 1, j).wait()
            pltpu.make_async_copy(
                ei_hbm.at[0, pl.ds(rbase, JJ)], S[b ^ 1], isem).wait()
            pltpu.make_async_copy(
                ei_hbm.at[1, pl.ds(rbase, JJ)], D[b], isem).wait()
            for j in range(JJ):
                gather_copy(b ^ 1, j).start()
            pltpu.async_copy(
                ei_hbm.at[0, pl.ds(jnp.minimum(rbase + 2 * JJ, rmax), JJ)],
                S[b], isem)
            pltpu.async_copy(
                ei_hbm.at[1, pl.ds(jnp.minimum(rbase + JJ, rmax), JJ)],
                D[b ^ 1], isem)
            for j in range(JJ):
                scatter_copy(b, j).start(add=True)
        return carry
    lax.fori_loop(0, npairs, pair_body, 0)

    for j in range(JJ):
        gather_copy(0, j).wait()
    for j in range(JJ):
        scatter_copy(1, j).wait()
    pltpu.make_async_copy(ei_hbm.at[0, pl.ds(0, JJ)], s1, isem).wait()
    pltpu.make_async_copy(ei_hbm.at[1, pl.ds(0, JJ)], d0, isem).wait()
    plsc.subcore_barrier()

    pltpu.sync_copy(acc_sh.at[pl.ds(row0, RPT)],
                    out_hbm.at[pl.ds(c * NPAD + row0, RPT)])


def kernel(r, edge_index, W1, b1, W2, b2, W3, b3):
    eye8 = jnp.eye(8, dtype=jnp.float32)
    g = jnp.kron(eye8, jnp.ones((NUM_TYPES, NUM_TYPES), jnp.float32))
    w2s = jnp.kron(eye8, W2)
    b2s = jnp.tile(b2, (8,)).reshape(1, 8 * NUM_TYPES)
    w3s = jnp.kron(eye8, W3)
    b3t = jnp.tile(b3, (8,)).reshape(1, 8 * OUT_DIM)
    p, ei = _compute_p(r, W1, b1.reshape(1, OUT_DIM), w2s, b2s, g,
                       edge_index)
    acc = _sc_segsum(p.reshape(-1, NUM_TYPES), ei)
    accp = acc.reshape(2 * NPAD // 8, 128)
    return _compute_out(accp, accp, g, w3s, b3t)


# speedup vs baseline: 1.0098x; 1.0098x over previous
import functools

import jax
import jax.numpy as jnp
from jax import lax
from jax.experimental import pallas as pl
from jax.experimental.pallas import tpu as pltpu
from jax.experimental.pallas import tpu_sc as plsc

N = 100000
E = 3200000
OUT_DIM = 128
NUM_TYPES = 16

NW = 32
CHUNK = 512
JJ = CHUNK // 128
EROWS = E // 128
NCHT = E // CHUNK
PAIRS = NCHT // 2
BASE_PAIRS = PAIRS // NW
EXTRA_PAIRS = PAIRS - BASE_PAIRS * NW

NPAD = 112000
RPT = NPAD // 16
ZR = 56
NZC = RPT // ZR

RB1 = 6400
GRID1 = -(-N // RB1)
NP8 = NPAD // 8
RB3P = 560
RB3 = RB3P * 8
GRID3 = -(-(N // 8) // RB3P)


def _p_body(r_ref, w1_ref, b1_ref, w2s_ref, b2s_ref, g_ref, p_ref):
    z = jnp.maximum(
        jnp.dot(r_ref[...], w1_ref[...], preferred_element_type=jnp.float32)
        + b1_ref[...], 0.0)
    zp = z.reshape(RB1 // 8, 8 * OUT_DIM)
    l = (jnp.dot(zp, w2s_ref[...], preferred_element_type=jnp.float32)
         + b2s_ref[...])
    m = jnp.max(l, axis=-1, keepdims=True)
    e = jnp.exp(jnp.maximum(l - m, -60.0))
    s = jnp.dot(e, g_ref[...], preferred_element_type=jnp.float32)
    p_ref[...] = e / s


def _out_body(a0_ref, a1_ref, g_ref, w3s_ref, b3t_ref, o_ref):
    sums = a0_ref[...] + a1_ref[...]
    rs = jnp.dot(sums, g_ref[...], preferred_element_type=jnp.float32)
    inv = jnp.where(rs > 0.5, 1.0 / jnp.maximum(rs, 0.5), 0.0)
    nd = sums * inv
    res = jnp.maximum(
        jnp.dot(nd, w3s_ref[...], preferred_element_type=jnp.float32)
        + b3t_ref[...], 0.0)
    o_ref[...] = res.reshape(RB3, 128)


_compute_p = pl.pallas_call(
    _p_body,
    grid=(GRID1,),
    in_specs=[
        pl.BlockSpec((RB1, NUM_TYPES), lambda i: (i, 0)),
        pl.BlockSpec((NUM_TYPES, OUT_DIM), lambda i: (0, 0)),
        pl.BlockSpec((1, OUT_DIM), lambda i: (0, 0)),
        pl.BlockSpec((8 * OUT_DIM, 8 * NUM_TYPES), lambda i: (0, 0)),
        pl.BlockSpec((1, 8 * NUM_TYPES), lambda i: (0, 0)),
        pl.BlockSpec((8 * NUM_TYPES, 8 * NUM_TYPES), lambda i: (0, 0)),
    ],
    out_specs=pl.BlockSpec((RB1 // 8, 8 * NUM_TYPES), lambda i: (i, 0)),
    out_shape=jax.ShapeDtypeStruct((N // 8, 8 * NUM_TYPES), jnp.float32),
)

_compute_out = pl.pallas_call(
    _out_body,
    grid=(GRID3,),
    in_specs=[
        pl.BlockSpec((RB3P, 128), lambda i: (i, 0)),
        pl.BlockSpec((RB3P, 128), lambda i: (i + NPAD // RB3, 0)),
        pl.BlockSpec((128, 128), lambda i: (0, 0)),
        pl.BlockSpec((128, 8 * OUT_DIM), lambda i: (0, 0)),
        pl.BlockSpec((1, 8 * OUT_DIM), lambda i: (0, 0)),
    ],
    out_specs=pl.BlockSpec((RB3, OUT_DIM), lambda i: (i, 0)),
    out_shape=jax.ShapeDtypeStruct((N, OUT_DIM), jnp.float32),
)


@functools.partial(
    pl.kernel,
    out_type=jax.ShapeDtypeStruct((2 * NPAD, NUM_TYPES), jnp.float32),
    mesh=plsc.VectorSubcoreMesh(core_axis_name="c", subcore_axis_name="s"),
    scratch_types=[
        pltpu.VMEM_SHARED((NPAD, NUM_TYPES), jnp.float32),
        pltpu.VMEM((JJ, 128), jnp.int32),
        pltpu.VMEM((JJ, 128), jnp.int32),
        pltpu.VMEM((JJ, 128), jnp.int32),
        pltpu.VMEM((JJ, 128), jnp.int32),
        pltpu.VMEM((CHUNK, NUM_TYPES), jnp.float32),
        pltpu.VMEM((CHUNK, NUM_TYPES), jnp.float32),
        pltpu.SemaphoreType.DMA,
        pltpu.SemaphoreType.DMA,
        pltpu.SemaphoreType.DMA,
        pltpu.SemaphoreType.DMA,
    ],
    compiler_params=pltpu.CompilerParams(use_tc_tiling_on_sc=False),
)
def _sc_segsum(p_hbm, ei_hbm, out_hbm,
               acc_sh, s0, s1, d0, d1, rb0, rb1,
               gsem0, gsem1, isem, ssem):
    c = lax.axis_index("c")
    s = lax.axis_index("s")
    wid = c * 16 + s
    S = (s0, s1)
    D = (d0, d1)
    RBUF = (rb0, rb1)
    GSEM = (gsem0, gsem1)

    zv = jnp.zeros((NUM_TYPES,), jnp.float32)
    for i in range(ZR):
        rb0[i] = zv
    row0 = s * RPT

    def zbody(k, carry):
        pltpu.sync_copy(rb0.at[pl.ds(0, ZR)],
                        acc_sh.at[pl.ds(row0 + k * ZR, ZR)])
        return carry
    lax.fori_loop(0, NZC, zbody, 0)
    plsc.subcore_barrier()

    pair0 = wid * BASE_PAIRS + jnp.minimum(wid, EXTRA_PAIRS)
    npairs = jnp.where(wid < EXTRA_PAIRS, BASE_PAIRS + 1, BASE_PAIRS)
    wrow = pair0 * 2 * JJ
    rmax = EROWS - JJ

    def gather_copy(b, j):
        return pltpu.make_async_copy(
            p_hbm.at[S[b].at[j]],
            RBUF[b].at[pl.ds(j * 128, 128)], GSEM[b])

    def scatter_copy(b, j):
        return pltpu.make_async_copy(
            RBUF[b].at[pl.ds(j * 128, 128)],
            acc_sh.at[D[b].at[j]], ssem)

    pltpu.sync_copy(ei_hbm.at[0, pl.ds(wrow, JJ)], s0)
    for j in range(JJ):
        gather_copy(0, j).start()
    pltpu.async_copy(ei_hbm.at[0, pl.ds(wrow + JJ, JJ)], s1, isem)
    pltpu.async_copy(ei_hbm.at[1, pl.ds(wrow, JJ)], d0, isem)

    def pair_body(u, carry):
        for b in range(2):
            rbase = wrow + (u * 2 + b) * JJ
            for j in range(JJ):
                gather_copy(b, j).wait()
            pltpu.make_async_copy(
                ei_hbm.at[0, pl.ds(rbase, JJ)], S[b ^ 1], isem).wait()
            pltpu.make_async_copy(
                ei_hbm.at[1, pl.ds(rbase, JJ)], D[b], isem).wait()
            for j in range(JJ):
                gather_copy(b ^ 1, j).start()
            pltpu.async_copy(
                ei_hbm.at[0, pl.ds(jnp.minimum(rbase + 2 * JJ, rmax), JJ)],
                S[b], isem)
            pltpu.async_copy(
                ei_hbm.at[1, pl.ds(jnp.minimum(rbase + JJ, rmax), JJ)],
                D[b ^ 1], isem)
            for j in range(JJ):
                scatter_copy(b, j).start(add=True)
            for j in range(JJ):
                scatter_copy(b, j).wait()
        return carry
    lax.fori_loop(0, npairs, pair_body, 0)

    for j in range(JJ):
        gather_copy(0, j).wait()
    pltpu.make_async_copy(ei_hbm.at[0, pl.ds(0, JJ)], s1, isem).wait()
    pltpu.make_async_copy(ei_hbm.at[1, pl.ds(0, JJ)], d0, isem).wait()
    plsc.subcore_barrier()

    pltpu.sync_copy(acc_sh.at[pl.ds(row0, RPT)],
                    out_hbm.at[pl.ds(c * NPAD + row0, RPT)])


def kernel(r, edge_index, W1, b1, W2, b2, W3, b3):
    eye8 = jnp.eye(8, dtype=jnp.float32)
    g = jnp.kron(eye8, jnp.ones((NUM_TYPES, NUM_TYPES), jnp.float32))
    w2s = jnp.kron(eye8, W2)
    b2s = jnp.tile(b2, (8,)).reshape(1, 8 * NUM_TYPES)
    w3s = jnp.kron(eye8, W3)
    b3t = jnp.tile(b3, (8,)).reshape(1, 8 * OUT_DIM)
    p = _compute_p(r, W1, b1.reshape(1, OUT_DIM), w2s, b2s, g)
    ei = edge_index.reshape(2, EROWS, 128)
    acc = _sc_segsum(p.reshape(-1, NUM_TYPES), ei)
    accp = acc.reshape(2 * NPAD // 8, 128)
    return _compute_out(accp, accp, g, w3s, b3t)
